# baseline (speedup 1.0000x reference)
import functools

import jax
import jax.numpy as jnp
from jax import lax
from jax.experimental import pallas as pl
from jax.experimental.pallas import tpu as pltpu
from jax.experimental.pallas import tpu_sc as plsc

NC, NS, L = 2, 16, 16
NW = NC * NS
B, S, H = 32, 1024, 1024
V = 512
SW = S // NW
R = 16
NKK = 2 * B


def _emb_body(x_hbm, pix_hbm, pos_hbm, out_hbm,
              idx_v, pos_v, g0, g1, o0, o1,
              gsem0, gsem1, stsem0, stsem1):
    wid = lax.axis_index("s") * NC + lax.axis_index("c")
    col0 = pl.multiple_of(wid * SW, SW)
    xblk = pl.multiple_of((wid // 4) * 128, 128)
    coff = (wid % 4) * SW
    G = (g0, g1)
    O = (o0, o1)
    GSEM = (gsem0, gsem1)
    STSEM = (stsem0, stsem1)

    pltpu.sync_copy(x_hbm.at[:, pl.ds(xblk, 128)], idx_v)
    pltpu.sync_copy(pos_hbm.at[pl.ds(col0, SW), :], pos_v)

    pltpu.async_copy(pix_hbm.at[idx_v.at[0, pl.ds(coff, R)]], g0, gsem0)
    pltpu.async_copy(pix_hbm.at[idx_v.at[0, pl.ds(coff + R, R)]], g1, gsem1)

    def step(kk, carry):
        b = kk // 2
        for slot in range(2):
            gbuf, obuf = G[slot], O[slot]
            srow = slot * R
            pltpu.make_async_copy(
                pix_hbm.at[idx_v.at[b, pl.ds(coff + srow, R)]], gbuf, GSEM[slot]
            ).wait()
            @pl.when(kk > 0)
            def _():
                pltpu.make_async_copy(
                    obuf, out_hbm.at[b, pl.ds(col0 + srow, R), :], STSEM[slot]
                ).wait()
            @plsc.parallel_loop(0, H // (2 * L), step=1, unroll=2)
            def _(u, _obuf=obuf, _gbuf=gbuf, _srow=srow):
                c0 = pl.ds(u * L, L)
                cl = pl.ds(u * 2 * L, L)
                ch = pl.ds(u * 2 * L + L, L)
                for r in range(R):
                    w = _gbuf[r, c0]
                    lo = jax.lax.bitcast_convert_type(w << 16, jnp.float32)
                    hi = jax.lax.bitcast_convert_type(
                        w & jnp.int32(-65536), jnp.float32)
                    _obuf[r, cl] = lo + pos_v[_srow + r, cl]
                    _obuf[r, ch] = hi + pos_v[_srow + r, ch]
            @pl.when(b + 1 < B)
            def _():
                pltpu.async_copy(
                    pix_hbm.at[idx_v.at[b + 1, pl.ds(coff + srow, R)]],
                    gbuf, GSEM[slot],
                )
            pltpu.async_copy(
                obuf, out_hbm.at[b, pl.ds(col0 + srow, R), :], STSEM[slot]
            )
        return carry

    lax.fori_loop(0, B, lambda i, c: step(2 * i, c), 0, unroll=False)

    for slot in range(2):
        pltpu.make_async_copy(
            O[slot],
            out_hbm.at[B - 1, pl.ds(col0 + slot * R, R), :],
            STSEM[slot],
        ).wait()


@jax.jit
def _emb(x, pix_table, pos_table):
    pix_bf = pix_table.astype(jnp.bfloat16)
    pix_bf = pix_bf.reshape(V, H // 32, 2, 16).transpose(0, 1, 3, 2)
    pix_bf = jax.lax.bitcast_convert_type(
        pix_bf.reshape(V, H // 2, 2), jnp.int32)
    run = pl.kernel(
        _emb_body,
        out_type=jax.ShapeDtypeStruct((B, S, H), jnp.float32),
        mesh=plsc.VectorSubcoreMesh(core_axis_name="c", subcore_axis_name="s"),
        scratch_types=[
            pltpu.VMEM((B, 128), jnp.int32),
            pltpu.VMEM((SW, H), jnp.float32),
            pltpu.VMEM((R, H // 2), jnp.int32),
            pltpu.VMEM((R, H // 2), jnp.int32),
            pltpu.VMEM((R, H), jnp.float32),
            pltpu.VMEM((R, H), jnp.float32),
            pltpu.SemaphoreType.DMA,
            pltpu.SemaphoreType.DMA,
            pltpu.SemaphoreType.DMA,
            pltpu.SemaphoreType.DMA,
        ],
    )
    return run(x, pix_bf, pos_table)


def kernel(x, pix_table, pos_table):
    return _emb(x, pix_table, pos_table)


# speedup vs baseline: 1.6490x; 1.6490x over previous
import jax
import jax.numpy as jnp
from jax import lax
from jax.experimental import pallas as pl
from jax.experimental.pallas import tpu as pltpu
from jax.experimental.pallas import tpu_sc as plsc

NC, NS, L = 2, 16, 16
NW = NC * NS
B, S, H = 32, 1024, 1024
SW = S // NW
R = 8
CPB = SW // R
NK = B * CPB


def _emb_body(x_hbm, pix_hbm, pos_hbm, out_hbm,
              idx_v, pos_v, g0, g1, g2, g3, o0, o1,
              gsem0, gsem1, gsem2, gsem3, stsem0, stsem1):
    wid = lax.axis_index("s") * NC + lax.axis_index("c")
    col0 = pl.multiple_of(wid * SW, SW)
    xblk = pl.multiple_of((wid // 4) * 128, 128)
    coff = (wid % 4) * SW
    G = (g0, g1, g2, g3)
    O = (o0, o1)
    GSEM = (gsem0, gsem1, gsem2, gsem3)
    STSEM = (stsem0, stsem1)

    pltpu.sync_copy(x_hbm.at[:, pl.ds(xblk, 128)], idx_v)
    pltpu.sync_copy(pos_hbm.at[pl.ds(col0, SW), :], pos_v)

    def start_gather(b, j):
        pltpu.async_copy(
            pix_hbm.at[idx_v.at[b, pl.ds(coff + j * R, R)]], G[j], GSEM[j]
        )

    def wait_gather(b, j):
        pltpu.make_async_copy(
            pix_hbm.at[idx_v.at[b, pl.ds(coff + j * R, R)]], G[j], GSEM[j]
        ).wait()

    def wait_store(b, j):
        pltpu.make_async_copy(
            O[j % 2], out_hbm.at[b, pl.ds(col0 + j * R, R), :], STSEM[j % 2]
        ).wait()

    for j in range(CPB):
        start_gather(0, j)

    def step(b, carry):
        for j in range(CPB):
            srow = j * R
            gbuf, obuf = G[j], O[j % 2]
            wait_gather(b, j)
            @pl.when((b > 0) | (j >= 2))
            def _():
                wait_store(b, j)
            @plsc.parallel_loop(0, H // L, step=1, unroll=2)
            def _(u, _obuf=obuf, _gbuf=gbuf, _srow=srow):
                cs = pl.ds(u * L, L)
                for r in range(R):
                    _obuf[r, cs] = _gbuf[r, cs] + pos_v[_srow + r, cs]
            @pl.when(b + 1 < B)
            def _():
                start_gather(b + 1, j)
            pltpu.async_copy(
                obuf, out_hbm.at[b, pl.ds(col0 + srow, R), :], STSEM[j % 2]
            )
        return carry

    lax.fori_loop(0, B, step, 0, unroll=False)

    for j in range(2, 4):
        wait_store(B - 1, j)


@jax.jit
def _emb(x, pix_table, pos_table):
    run = pl.kernel(
        _emb_body,
        out_type=jax.ShapeDtypeStruct((B, S, H), jnp.float32),
        mesh=plsc.VectorSubcoreMesh(core_axis_name="c", subcore_axis_name="s"),
        scratch_types=[
            pltpu.VMEM((B, 128), jnp.int32),
            pltpu.VMEM((SW, H), jnp.float32),
            pltpu.VMEM((R, H), jnp.float32),
            pltpu.VMEM((R, H), jnp.float32),
            pltpu.VMEM((R, H), jnp.float32),
            pltpu.VMEM((R, H), jnp.float32),
            pltpu.VMEM((R, H), jnp.float32),
            pltpu.VMEM((R, H), jnp.float32),
            pltpu.SemaphoreType.DMA,
            pltpu.SemaphoreType.DMA,
            pltpu.SemaphoreType.DMA,
            pltpu.SemaphoreType.DMA,
            pltpu.SemaphoreType.DMA,
            pltpu.SemaphoreType.DMA,
        ],
    )
    return run(x, pix_table, pos_table)


def kernel(x, pix_table, pos_table):
    return _emb(x, pix_table, pos_table)
